# Initial kernel scaffold; baseline (speedup 1.0000x reference)
import jax
import jax.numpy as jnp
from jax.experimental import pallas as pl


def kernel(ref, query):
    raise NotImplementedError("write your pallas kernel here")


# speedup vs baseline: 2.7638x; 2.7638x over previous
import jax
import jax.numpy as jnp
from jax import lax
from jax.experimental import pallas as pl
from jax.experimental.pallas import tpu as pltpu

_TOPK = 16
_NB = 2048


def _knn_body(q_ref, r_ref, q2_ref, r2_ref, dval_ref, didx_ref, rv_ref, ri_ref):
    n = pl.program_id(1)
    nblk = pl.num_programs(1)
    m_rows = q_ref.shape[1]

    @pl.when(n == 0)
    def _init():
        rv_ref[...] = jnp.full((m_rows, _TOPK), jnp.inf, jnp.float32)
        ri_ref[...] = jnp.zeros((m_rows, _TOPK), jnp.int32)

    q = q_ref[0]
    r = r_ref[0]
    q2 = q2_ref[0, 0]
    r2 = r2_ref[0, 0, 0]

    dot = lax.dot_general(q, r, (((1,), (1,)), ((), ())),
                          preferred_element_type=jnp.float32)
    d2 = (q2[:, None] + r2[None, :]) - 2.0 * dot
    dist = jnp.sqrt(jnp.maximum(d2, 0.0))

    ci = lax.broadcasted_iota(jnp.int32, dist.shape, 1)
    nb = dist.shape[1]

    def cond(c):
        k, cont, _, _, _, _ = c
        return (k < _TOPK) & cont

    def body(c):
        k, _, pm, pc, rv, ri = c
        after = (dist > pm) | ((dist == pm) & (ci > pc))
        masked = jnp.where(after, dist, jnp.inf)
        m = jnp.min(masked, axis=1, keepdims=True)
        c_sel = jnp.min(jnp.where(masked == m, ci, nb), axis=1,
                        keepdims=True)
        cont = jnp.any(m < rv[:, _TOPK - 1:_TOPK])
        gcol = c_sel + n * nb
        pos = jnp.sum((rv <= m).astype(jnp.int32), axis=1,
                      keepdims=True)
        slot = lax.broadcasted_iota(jnp.int32, rv.shape, 1)
        gt = slot >= pos
        ins_here = slot == pos
        sh_rv = jnp.concatenate([rv[:, :1], rv[:, :-1]], axis=1)
        sh_ri = jnp.concatenate([ri[:, :1], ri[:, :-1]], axis=1)
        new_rv = jnp.where(gt, jnp.where(ins_here,
                                         jnp.broadcast_to(m, rv.shape),
                                         sh_rv), rv)
        new_ri = jnp.where(gt, jnp.where(ins_here,
                                         jnp.broadcast_to(gcol, ri.shape),
                                         sh_ri), ri)
        return k + 1, cont, m, c_sel, new_rv, new_ri

    init = (jnp.int32(0), True,
            jnp.full((m_rows, 1), -jnp.inf, jnp.float32),
            jnp.full((m_rows, 1), -1, jnp.int32),
            rv_ref[...], ri_ref[...])
    _, _, _, _, rv, ri = lax.while_loop(cond, body, init)
    rv_ref[...] = rv
    ri_ref[...] = ri

    @pl.when(n == nblk - 1)
    def _emit():
        dval_ref[0] = rv
        didx_ref[0] = ri


def kernel(ref, query):
    B, N, D = ref.shape
    M = query.shape[1]
    nblk = -(-N // _NB)
    npad = nblk * _NB

    r2 = jnp.sum(ref * ref, axis=2)
    q2 = jnp.sum(query * query, axis=2)
    refp = ref
    r2p = r2
    if npad != N:
        refp = jnp.concatenate(
            [ref, jnp.zeros((B, npad - N, D), ref.dtype)], axis=1)
        r2p = jnp.concatenate(
            [r2, jnp.full((B, npad - N), 1e30, r2.dtype)], axis=1)
    r2p = r2p.reshape(B, nblk, 1, _NB)
    q2r = q2.reshape(B, 1, M)

    dval, didx = pl.pallas_call(
        _knn_body,
        grid=(B, nblk),
        in_specs=[
            pl.BlockSpec((1, M, D), lambda b, n: (b, 0, 0)),
            pl.BlockSpec((1, _NB, D), lambda b, n: (b, n, 0)),
            pl.BlockSpec((1, 1, M), lambda b, n: (b, 0, 0)),
            pl.BlockSpec((1, 1, 1, _NB), lambda b, n: (b, n, 0, 0)),
        ],
        out_specs=[
            pl.BlockSpec((1, M, _TOPK), lambda b, n: (b, 0, 0)),
            pl.BlockSpec((1, M, _TOPK), lambda b, n: (b, 0, 0)),
        ],
        out_shape=[
            jax.ShapeDtypeStruct((B, M, _TOPK), jnp.float32),
            jax.ShapeDtypeStruct((B, M, _TOPK), jnp.int32),
        ],
        scratch_shapes=[
            pltpu.VMEM((M, _TOPK), jnp.float32),
            pltpu.VMEM((M, _TOPK), jnp.int32),
        ],
        compiler_params=pltpu.CompilerParams(
            dimension_semantics=("arbitrary", "arbitrary")),
    )(query, refp, q2r, r2p)
    return dval, didx.astype(jnp.int64)
